# baseline (speedup 1.0000x reference)
import functools

import jax
import jax.numpy as jnp
from jax import lax
from jax.experimental import pallas as pl
from jax.experimental.pallas import tpu as pltpu
from jax.experimental.pallas import tpu_sc as plsc

B = 16384
MAX_LEN = 50
VOCAB = 1000000
EMBED = 32

N = B * MAX_LEN
NW = 32
NPW = N // NW
BCH = 32
CH = BCH * MAX_LEN
NCHUNK = NPW // CH
BPW = B // NW

_mesh = plsc.VectorSubcoreMesh(core_axis_name="c", subcore_axis_name="s")


@functools.partial(
    pl.kernel,
    mesh=_mesh,
    compiler_params=pltpu.CompilerParams(use_tc_tiling_on_sc=False),
    out_type=jax.ShapeDtypeStruct((B, MAX_LEN, EMBED), jnp.float32),
    scratch_types=[
        pltpu.VMEM((BCH, MAX_LEN), jnp.int32),
        pltpu.VMEM((BCH, MAX_LEN, EMBED), jnp.float32),
        pltpu.VMEM((BPW + 16,), jnp.int32),
        pltpu.SemaphoreType.DMA,
    ],
)
def _embed_sc(idx_hbm, seq_hbm, table_hbm, out_hbm, idx_v, rows_v, seq_v, sem):
    wid = lax.axis_index("s") * 2 + lax.axis_index("c")
    base = wid * NPW
    pltpu.sync_copy(seq_hbm.at[pl.ds(wid * BPW, BPW)], seq_v.at[pl.ds(0, BPW)])
    zvec = jnp.zeros((16,), jnp.float32)

    def chunk_body(g, _):
        boff = wid * BPW + g * BCH
        pltpu.sync_copy(idx_hbm.at[pl.ds(boff, BCH)], idx_v)
        copies = [
            pltpu.async_copy(
                table_hbm.at[idx_v.at[brel]],
                rows_v.at[brel],
                sem,
            )
            for brel in range(BCH)
        ]
        for c in copies:
            c.wait()

        def zero_b(brel, _):
            sl = seq_v[pl.ds(g * BCH + brel, 16)][0]

            def zero_row(r, _):
                rows_v[brel, r, pl.ds(0, 16)] = zvec
                rows_v[brel, r, pl.ds(16, 16)] = zvec
                return 0

            lax.fori_loop(sl, MAX_LEN, zero_row, 0)
            return 0

        lax.fori_loop(0, BCH, zero_b, 0)

        pltpu.sync_copy(rows_v, out_hbm.at[pl.ds(boff, BCH)])
        return 0

    lax.fori_loop(0, NCHUNK, chunk_body, 0)


def kernel(indices, seq_lens, table):
    idx = indices.astype(jnp.int32)
    seq = seq_lens.astype(jnp.int32)
    out = _embed_sc(idx, seq, table)
    return out.reshape(B, MAX_LEN * EMBED)


# speedup vs baseline: 1.3940x; 1.3940x over previous
import functools

import jax
import jax.numpy as jnp
from jax import lax
from jax.experimental import pallas as pl
from jax.experimental.pallas import tpu as pltpu
from jax.experimental.pallas import tpu_sc as plsc

B = 16384
MAX_LEN = 50
VOCAB = 1000000
EMBED = 32

N = B * MAX_LEN
NW = 32
NPW = N // NW
BCH = 32
CH = BCH * MAX_LEN
NCHUNK = NPW // CH
BPW = B // NW

_mesh = plsc.VectorSubcoreMesh(core_axis_name="c", subcore_axis_name="s")


@functools.partial(
    pl.kernel,
    mesh=_mesh,
    compiler_params=pltpu.CompilerParams(use_tc_tiling_on_sc=False),
    out_type=jax.ShapeDtypeStruct((N, EMBED), jnp.float32),
    scratch_types=[
        pltpu.VMEM((NPW,), jnp.int32),
        pltpu.VMEM((2, CH, EMBED), jnp.float32),
        pltpu.VMEM((BPW + 16,), jnp.int32),
        pltpu.SemaphoreType.DMA,
        pltpu.SemaphoreType.DMA,
        pltpu.SemaphoreType.DMA,
    ],
)
def _embed_sc(
    idx_hbm, seq_hbm, table_hbm, out_hbm, idx_v, rows_v, seq_v, gsem, osem0, osem1
):
    wid = lax.axis_index("s") * 2 + lax.axis_index("c")
    base = wid * NPW
    pltpu.sync_copy(seq_hbm.at[pl.ds(wid * BPW, BPW)], seq_v.at[pl.ds(0, BPW)])
    pltpu.sync_copy(idx_hbm.at[pl.ds(base, NPW)], idx_v)
    zvec = jnp.zeros((16,), jnp.float32)
    osems = (osem0, osem1)

    def fire_gather(g):
        return pltpu.async_copy(
            table_hbm.at[idx_v.at[pl.ds(g * CH, CH)]],
            rows_v.at[(g % 2)],
            gsem,
        )

    def zero_chunk(g):
        s = g % 2

        def zero_b(brel, _):
            sl = seq_v[pl.ds(g * BCH + brel, 16)][0]

            def zero_row(r, _):
                rows_v[s, brel * MAX_LEN + r, pl.ds(0, 16)] = zvec
                rows_v[s, brel * MAX_LEN + r, pl.ds(16, 16)] = zvec
                return 0

            lax.fori_loop(sl, MAX_LEN, zero_row, 0)
            return 0

        lax.fori_loop(0, BCH, zero_b, 0)

    out_copies = [None, None]
    gather = fire_gather(0)
    for g in range(NCHUNK):
        s = g % 2
        if g + 1 < NCHUNK:
            if out_copies[1 - s] is not None:
                out_copies[1 - s].wait()
            next_gather = fire_gather(g + 1)
        gather.wait()
        zero_chunk(g)
        out_copies[s] = pltpu.async_copy(
            rows_v.at[s], out_hbm.at[pl.ds(base + g * CH, CH)], osems[s]
        )
        if g + 1 < NCHUNK:
            gather = next_gather
    for c in out_copies:
        if c is not None:
            c.wait()


def kernel(indices, seq_lens, table):
    idx = indices.astype(jnp.int32).reshape(-1)
    seq = seq_lens.astype(jnp.int32)
    out = _embed_sc(idx, seq, table)
    return out.reshape(B, MAX_LEN * EMBED)
